# Initial kernel scaffold; baseline (speedup 1.0000x reference)
import jax
import jax.numpy as jnp
from jax.experimental import pallas as pl


def kernel(x, W1s, W1n, b1, g1, be1, W2s, W2n, b2, g2, be2, Wd1, bd1, g3, be3, Wd2, bd2, edge_index, membership):
    raise NotImplementedError("write your pallas kernel here")


# speedup vs baseline: 6.1962x; 6.1962x over previous
import functools

import jax
import jax.numpy as jnp
from jax import lax
from jax.experimental import pallas as pl
from jax.experimental.pallas import tpu as pltpu
from jax.experimental.pallas import tpu_sc as plsc

N = 10000
E = 320000
D = 128
B = 50
NT = 12

NC = 2
NS = 16
NW = NC * NS
EW = E // NW
CHUNK = 80
NCHUNK = EW // CHUNK
NBLK = 5
CPB = NCHUNK // NBLK
RPT = 624
TAIL = N - NS * RPT
BP = 64

def _edge_agg_body(zeros_hbm, src_hbm, dst_hbm, x_hbm, out_hbm,
                   agg_sh, src_v, dst_v, rows_a, rows_b, sem_a, sem_b):
    c = lax.axis_index("c")
    s = lax.axis_index("s")
    wid = s * NC + c
    rbase = s * RPT

    pltpu.sync_copy(zeros_hbm.at[pl.ds(rbase, RPT)],
                    agg_sh.at[pl.ds(rbase, RPT)])
    @pl.when(s == NS - 1)
    def _():
        pltpu.sync_copy(zeros_hbm.at[pl.ds(NS * RPT, TAIL)],
                        agg_sh.at[pl.ds(NS * RPT, TAIL)])
    plsc.subcore_barrier()

    def blk_body(blk, _):
        pltpu.sync_copy(src_hbm.at[wid, blk], src_v)
        pltpu.sync_copy(dst_hbm.at[wid, blk], dst_v)

        def body(i, _):
            j0 = 2 * i
            j1 = 2 * i + 1
            cp_a = pltpu.async_copy(x_hbm.at[src_v.at[j0]], rows_a, sem_a)
            cp_b = pltpu.async_copy(x_hbm.at[src_v.at[j1]], rows_b, sem_b)
            cp_a.wait()
            pltpu.sync_copy(rows_a, agg_sh.at[dst_v.at[j0]], add=True)
            cp_b.wait()
            pltpu.sync_copy(rows_b, agg_sh.at[dst_v.at[j1]], add=True)
            return 0

        lax.fori_loop(0, CPB // 2, body, 0)
        pltpu.async_copy(x_hbm.at[src_v.at[CPB - 1]], rows_a, sem_a).wait()
        pltpu.sync_copy(rows_a, agg_sh.at[dst_v.at[CPB - 1]], add=True)
        return 0

    lax.fori_loop(0, NBLK, blk_body, 0)
    plsc.subcore_barrier()

    pltpu.sync_copy(agg_sh.at[pl.ds(rbase, RPT)],
                    out_hbm.at[c, pl.ds(rbase, RPT)])
    @pl.when(s == NS - 1)
    def _():
        pltpu.sync_copy(agg_sh.at[pl.ds(NS * RPT, TAIL)],
                        out_hbm.at[c, pl.ds(NS * RPT, TAIL)])


@functools.cache
def _edge_agg_call():
    mesh = plsc.VectorSubcoreMesh(
        core_axis_name="c", subcore_axis_name="s",
        num_cores=NC, num_subcores=NS)
    return pl.kernel(
        _edge_agg_body,
        out_type=jax.ShapeDtypeStruct((NC, N, D), jnp.float32),
        mesh=mesh,
        scratch_types=[
            pltpu.VMEM_SHARED((N, D), jnp.float32),
            pltpu.VMEM((CPB, CHUNK), jnp.int32),
            pltpu.VMEM((CPB, CHUNK), jnp.int32),
            pltpu.VMEM((CHUNK, D), jnp.float32),
            pltpu.VMEM((CHUNK, D), jnp.float32),
            pltpu.SemaphoreType.DMA,
            pltpu.SemaphoreType.DMA,
        ],
    )


def _conv_body(x_ref, agg0_ref, agg1_ref, w1s_ref, w1n_ref, b1_ref, g1_ref,
               be1_ref, hb1_ref):
    f32 = jnp.float32
    x = x_ref[...]
    agg = agg0_ref[...] + agg1_ref[...]
    h1 = jnp.tanh(
        jax.lax.dot(x, w1s_ref[...], precision=lax.Precision.HIGHEST, preferred_element_type=f32)
        + jax.lax.dot(agg, w1n_ref[...], precision=lax.Precision.HIGHEST, preferred_element_type=f32)
        + b1_ref[...])
    m = jnp.mean(h1, axis=0, keepdims=True)
    v = jnp.mean((h1 - m) ** 2, axis=0, keepdims=True)
    hb1_ref[...] = g1_ref[...] * (h1 - m) * jax.lax.rsqrt(v + 1e-3) + be1_ref[...]


_conv_call = pl.pallas_call(
    _conv_body,
    out_shape=jax.ShapeDtypeStruct((N, D), jnp.float32),
)


def _readout_body(memrow_ref, memcol_ref, wd2e_ref, wd2o_ref, bd2e_ref,
                  bd2o_ref, hb1_ref, p0_ref, p1_ref, gmax_ref):
    f32 = jnp.float32
    hb1 = hb1_ref[...]

    ids = lax.broadcasted_iota(jnp.int32, (BP, N), 0)
    oht = jnp.where(ids == memrow_ref[...], f32(1.0), f32(0.0))
    gsum = jax.lax.dot(oht, hb1, precision=lax.Precision.HIGHEST, preferred_element_type=f32)

    neg_inf = f32(-jnp.inf)
    def gmax_body(b, _):
        mask = memcol_ref[...] == b
        seg = jnp.where(mask, hb1_ref[...], neg_inf)
        gmax_ref[pl.ds(b, 1), :] = jnp.max(seg, axis=0, keepdims=True)
        return 0
    lax.fori_loop(0, B, gmax_body, 0)
    gmax = gmax_ref[...]

    ro = jnp.tanh(jnp.concatenate([gsum, gmax], axis=1))
    le = jax.lax.dot(ro, wd2e_ref[...], precision=lax.Precision.HIGHEST, preferred_element_type=f32) + bd2e_ref[...]
    lo = jax.lax.dot(ro, wd2o_ref[...], precision=lax.Precision.HIGHEST, preferred_element_type=f32) + bd2o_ref[...]
    mx = jnp.maximum(le, lo)
    ee = jnp.exp(le - mx)
    eo = jnp.exp(lo - mx)
    inv = 1.0 / (ee + eo)
    p0_ref[...] = ee * inv
    p1_ref[...] = eo * inv


_readout_call = pl.pallas_call(
    _readout_body,
    out_shape=[
        jax.ShapeDtypeStruct((BP, NT), jnp.float32),
        jax.ShapeDtypeStruct((BP, NT), jnp.float32),
    ],
    scratch_shapes=[
        pltpu.VMEM((BP, D), jnp.float32),
    ],
)


def kernel(x, W1s, W1n, b1, g1, be1, W2s, W2n, b2, g2, be2, Wd1, bd1, g3,
           be3, Wd2, bd2, edge_index, membership):
    src2d = edge_index[0].reshape(NW, NBLK, CPB, CHUNK)
    dst2d = edge_index[1].reshape(NW, NBLK, CPB, CHUNK)
    zeros = jnp.zeros((N, D), jnp.float32)
    parts = _edge_agg_call()(zeros, src2d, dst2d, x)

    hb1 = _conv_call(x, parts[0], parts[1], W1s, W1n, b1.reshape(1, D),
                     g1.reshape(1, D), be1.reshape(1, D))
    memrow = membership.reshape(1, N)
    memcol = membership.reshape(N, 1)
    p0, p1 = _readout_call(
        memrow, memcol, Wd2[:, 0::2], Wd2[:, 1::2],
        bd2[0::2].reshape(1, NT), bd2[1::2].reshape(1, NT), hb1)
    return jnp.stack([p0[:B], p1[:B]], axis=-1)


# speedup vs baseline: 7.3027x; 1.1786x over previous
import functools

import jax
import jax.numpy as jnp
from jax import lax
from jax.experimental import pallas as pl
from jax.experimental.pallas import tpu as pltpu
from jax.experimental.pallas import tpu_sc as plsc

N = 10000
E = 320000
D = 128
B = 50
NT = 12

NC = 2
NS = 16
NW = NC * NS
EW = E // NW
CHUNK = 50
NCHUNK = EW // CHUNK
NBLK = 5
CPB = NCHUNK // NBLK
RING = 4
NGRP = CPB // RING
RPT = 624
TAIL = N - NS * RPT
BP = 64
HIGH = lax.Precision.HIGHEST


def _edge_agg_body(zeros_hbm, src_hbm, dst_hbm, x_hbm, out0_hbm, out1_hbm,
                   agg_sh, src_v, dst_v, r0, r1, r2, r3,
                   g0, g1, g2, g3, s0, s1, s2, s3):
    rows = [r0, r1, r2, r3]
    gsem = [g0, g1, g2, g3]
    ssem = [s0, s1, s2, s3]
    c = lax.axis_index("c")
    s = lax.axis_index("s")
    wid = s * NC + c
    rbase = s * RPT

    pltpu.sync_copy(zeros_hbm.at[pl.ds(rbase, RPT)],
                    agg_sh.at[pl.ds(rbase, RPT)])
    @pl.when(s == NS - 1)
    def _():
        pltpu.sync_copy(zeros_hbm.at[pl.ds(NS * RPT, TAIL)],
                        agg_sh.at[pl.ds(NS * RPT, TAIL)])
    plsc.subcore_barrier()

    def blk_body(blk, _):
        pltpu.sync_copy(src_hbm.at[wid, blk], src_v)
        pltpu.sync_copy(dst_hbm.at[wid, blk], dst_v)
        for k in range(RING):
            pltpu.async_copy(x_hbm.at[src_v.at[k]], rows[k], gsem[k])

        def grp(g, _):
            scat = []
            for k in range(RING):
                j = RING * g + k
                pltpu.make_async_copy(
                    x_hbm.at[src_v.at[j]], rows[k], gsem[k]).wait()
                scat.append(pltpu.async_copy(
                    rows[k], agg_sh.at[dst_v.at[j]], ssem[k], add=True))
            for k in range(RING):
                scat[k].wait()
                @pl.when(g < NGRP - 1)
                def _():
                    jn = RING * g + k + RING
                    pltpu.async_copy(x_hbm.at[src_v.at[jn]], rows[k], gsem[k])
            return 0

        lax.fori_loop(0, NGRP, grp, 0)
        return 0

    lax.fori_loop(0, NBLK, blk_body, 0)
    plsc.subcore_barrier()

    @pl.when(c == 0)
    def _():
        pltpu.sync_copy(agg_sh.at[pl.ds(rbase, RPT)],
                        out0_hbm.at[pl.ds(rbase, RPT)])
        @pl.when(s == NS - 1)
        def _():
            pltpu.sync_copy(agg_sh.at[pl.ds(NS * RPT, TAIL)],
                            out0_hbm.at[pl.ds(NS * RPT, TAIL)])
    @pl.when(c == 1)
    def _():
        pltpu.sync_copy(agg_sh.at[pl.ds(rbase, RPT)],
                        out1_hbm.at[pl.ds(rbase, RPT)])
        @pl.when(s == NS - 1)
        def _():
            pltpu.sync_copy(agg_sh.at[pl.ds(NS * RPT, TAIL)],
                            out1_hbm.at[pl.ds(NS * RPT, TAIL)])


@functools.cache
def _edge_agg_call():
    mesh = plsc.VectorSubcoreMesh(
        core_axis_name="c", subcore_axis_name="s",
        num_cores=NC, num_subcores=NS)
    return pl.kernel(
        _edge_agg_body,
        out_type=[jax.ShapeDtypeStruct((N, D), jnp.float32),
                  jax.ShapeDtypeStruct((N, D), jnp.float32)],
        mesh=mesh,
        scratch_types=(
            [pltpu.VMEM_SHARED((N, D), jnp.float32)]
            + [pltpu.VMEM((CPB, CHUNK), jnp.int32)] * 2
            + [pltpu.VMEM((CHUNK, D), jnp.float32)] * RING
            + [pltpu.SemaphoreType.DMA] * (2 * RING)
        ),
    )


def _xw_body(x_ref, w1s_ref, xw_ref):
    xw_ref[...] = jax.lax.dot(x_ref[...], w1s_ref[...], precision=HIGH,
                              preferred_element_type=jnp.float32)


_xw_call = pl.pallas_call(
    _xw_body, out_shape=jax.ShapeDtypeStruct((N, D), jnp.float32))


def _conv_body(xw_ref, agg0_ref, agg1_ref, w1n_ref, b1_ref, g1_ref,
               be1_ref, hb1_ref):
    agg = agg0_ref[...] + agg1_ref[...]
    h1 = jnp.tanh(
        xw_ref[...]
        + jax.lax.dot(agg, w1n_ref[...], precision=HIGH,
                      preferred_element_type=jnp.float32)
        + b1_ref[...])
    m = jnp.mean(h1, axis=0, keepdims=True)
    v = jnp.mean((h1 - m) ** 2, axis=0, keepdims=True)
    hb1_ref[...] = g1_ref[...] * (h1 - m) * jax.lax.rsqrt(v + 1e-3) + be1_ref[...]


_conv_call = pl.pallas_call(
    _conv_body, out_shape=jax.ShapeDtypeStruct((N, D), jnp.float32))


def _readout_body(memrow_ref, memcol_ref, wd2e_ref, wd2o_ref, bd2e_ref,
                  bd2o_ref, hb1_ref, p0_ref, p1_ref, gmax_ref):
    f32 = jnp.float32
    hb1 = hb1_ref[...]

    ids = lax.broadcasted_iota(jnp.int32, (BP, N), 0)
    oht = jnp.where(ids == memrow_ref[...], f32(1.0), f32(0.0))
    gsum = jax.lax.dot(oht, hb1, precision=HIGH, preferred_element_type=f32)

    neg_inf = f32(-jnp.inf)
    def gmax_body(b, _):
        mask = memcol_ref[...] == b
        seg = jnp.where(mask, hb1_ref[...], neg_inf)
        gmax_ref[pl.ds(b, 1), :] = jnp.max(seg, axis=0, keepdims=True)
        return 0
    lax.fori_loop(0, B, gmax_body, 0)
    gmax = gmax_ref[...]

    ro = jnp.tanh(jnp.concatenate([gsum, gmax], axis=1))
    le = jax.lax.dot(ro, wd2e_ref[...], precision=HIGH,
                     preferred_element_type=f32) + bd2e_ref[...]
    lo = jax.lax.dot(ro, wd2o_ref[...], precision=HIGH,
                     preferred_element_type=f32) + bd2o_ref[...]
    mx = jnp.maximum(le, lo)
    ee = jnp.exp(le - mx)
    eo = jnp.exp(lo - mx)
    inv = 1.0 / (ee + eo)
    p0_ref[...] = ee * inv
    p1_ref[...] = eo * inv


_readout_call = pl.pallas_call(
    _readout_body,
    out_shape=[
        jax.ShapeDtypeStruct((BP, NT), jnp.float32),
        jax.ShapeDtypeStruct((BP, NT), jnp.float32),
    ],
    scratch_shapes=[
        pltpu.VMEM((BP, D), jnp.float32),
    ],
)


def kernel(x, W1s, W1n, b1, g1, be1, W2s, W2n, b2, g2, be2, Wd1, bd1, g3,
           be3, Wd2, bd2, edge_index, membership):
    src4d = edge_index[0].reshape(NW, NBLK, CPB, CHUNK)
    dst4d = edge_index[1].reshape(NW, NBLK, CPB, CHUNK)
    zeros = jnp.zeros((N, D), jnp.float32)
    agg0, agg1 = _edge_agg_call()(zeros, src4d, dst4d, x)

    xw = _xw_call(x, W1s)
    hb1 = _conv_call(xw, agg0, agg1, W1n, b1.reshape(1, D),
                     g1.reshape(1, D), be1.reshape(1, D))
    memrow = membership.reshape(1, N)
    memcol = membership.reshape(N, 1)
    p0, p1 = _readout_call(
        memrow, memcol, Wd2[:, 0::2], Wd2[:, 1::2],
        bd2[0::2].reshape(1, NT), bd2[1::2].reshape(1, NT), hb1)
    return jnp.stack([p0[:B], p1[:B]], axis=-1)


# speedup vs baseline: 8.2259x; 1.1264x over previous
import functools

import jax
import jax.numpy as jnp
from jax import lax
from jax.experimental import pallas as pl
from jax.experimental.pallas import tpu as pltpu
from jax.experimental.pallas import tpu_sc as plsc

N = 10000
E = 320000
D = 128
B = 50
NT = 12

NC = 2
NS = 16
NW = NC * NS
EW = E // NW
CHUNK = 50
NCHUNK = EW // CHUNK
NBLK = 5
CPB = NCHUNK // NBLK
RING = 4
NGRP = CPB // RING
RPT = 624
TAIL = N - NS * RPT
BP = 64
HIGH = lax.Precision.HIGHEST


def _edge_agg_body(zeros_hbm, src_hbm, dst_hbm, x_hbm, out0_hbm, out1_hbm,
                   agg_sh, src_v, dst_v, r0, r1, r2, r3,
                   g0, g1, g2, g3, s0, s1, s2, s3):
    rows = [r0, r1, r2, r3]
    gsem = [g0, g1, g2, g3]
    ssem = [s0, s1, s2, s3]
    c = lax.axis_index("c")
    s = lax.axis_index("s")
    wid = s * NC + c
    rbase = s * RPT

    pltpu.sync_copy(zeros_hbm.at[pl.ds(rbase, RPT)],
                    agg_sh.at[pl.ds(rbase, RPT)])
    @pl.when(s == NS - 1)
    def _():
        pltpu.sync_copy(zeros_hbm.at[pl.ds(NS * RPT, TAIL)],
                        agg_sh.at[pl.ds(NS * RPT, TAIL)])
    plsc.subcore_barrier()

    def blk_body(blk, _):
        pltpu.sync_copy(src_hbm.at[wid, blk], src_v)
        pltpu.sync_copy(dst_hbm.at[wid, blk], dst_v)
        for k in range(RING):
            pltpu.async_copy(x_hbm.at[src_v.at[k]], rows[k], gsem[k])

        def grp(g, _):
            scat = []
            for k in range(RING):
                j = RING * g + k
                pltpu.make_async_copy(
                    x_hbm.at[src_v.at[j]], rows[k], gsem[k]).wait()
                scat.append(pltpu.async_copy(
                    rows[k], agg_sh.at[dst_v.at[j]], ssem[k], add=True))
            for k in range(RING):
                scat[k].wait()
                @pl.when(g < NGRP - 1)
                def _():
                    jn = RING * g + k + RING
                    pltpu.async_copy(x_hbm.at[src_v.at[jn]], rows[k], gsem[k])
            return 0

        lax.fori_loop(0, NGRP, grp, 0)
        return 0

    lax.fori_loop(0, NBLK, blk_body, 0)
    plsc.subcore_barrier()

    @pl.when(c == 0)
    def _():
        pltpu.sync_copy(agg_sh.at[pl.ds(rbase, RPT)],
                        out0_hbm.at[pl.ds(rbase, RPT)])
        @pl.when(s == NS - 1)
        def _():
            pltpu.sync_copy(agg_sh.at[pl.ds(NS * RPT, TAIL)],
                            out0_hbm.at[pl.ds(NS * RPT, TAIL)])
    @pl.when(c == 1)
    def _():
        pltpu.sync_copy(agg_sh.at[pl.ds(rbase, RPT)],
                        out1_hbm.at[pl.ds(rbase, RPT)])
        @pl.when(s == NS - 1)
        def _():
            pltpu.sync_copy(agg_sh.at[pl.ds(NS * RPT, TAIL)],
                            out1_hbm.at[pl.ds(NS * RPT, TAIL)])


@functools.cache
def _edge_agg_call():
    mesh = plsc.VectorSubcoreMesh(
        core_axis_name="c", subcore_axis_name="s",
        num_cores=NC, num_subcores=NS)
    return pl.kernel(
        _edge_agg_body,
        out_type=[jax.ShapeDtypeStruct((N, D), jnp.float32),
                  jax.ShapeDtypeStruct((N, D), jnp.float32)],
        mesh=mesh,
        scratch_types=(
            [pltpu.VMEM_SHARED((N, D), jnp.float32)]
            + [pltpu.VMEM((CPB, CHUNK), jnp.int32)] * 2
            + [pltpu.VMEM((CHUNK, D), jnp.float32)] * RING
            + [pltpu.SemaphoreType.DMA] * (2 * RING)
        ),
    )


def _xw_body(x_ref, w1s_ref, xw_ref):
    xw_ref[...] = jax.lax.dot(x_ref[...], w1s_ref[...], precision=HIGH,
                              preferred_element_type=jnp.float32)


_xw_call = pl.pallas_call(
    _xw_body, out_shape=jax.ShapeDtypeStruct((N, D), jnp.float32))


def _conv_body(xw_ref, agg0_ref, agg1_ref, w1n_ref, b1_ref, g1_ref,
               be1_ref, hb1_ref):
    agg = agg0_ref[...] + agg1_ref[...]
    h1 = jnp.tanh(
        xw_ref[...]
        + jax.lax.dot(agg, w1n_ref[...], precision=HIGH,
                      preferred_element_type=jnp.float32)
        + b1_ref[...])
    m = jnp.mean(h1, axis=0, keepdims=True)
    v = jnp.mean((h1 - m) ** 2, axis=0, keepdims=True)
    hb1_ref[...] = g1_ref[...] * (h1 - m) * jax.lax.rsqrt(v + 1e-3) + be1_ref[...]


_conv_call = pl.pallas_call(
    _conv_body, out_shape=jax.ShapeDtypeStruct((N, D), jnp.float32))


def _readout_body(memrow_ref, memblk_ref, mems_ref, wd2e_ref, wd2o_ref,
                  bd2e_ref, bd2o_ref, hb1_ref, p0_ref, p1_ref, gmax_ref,
                  bmax_ref):
    f32 = jnp.float32
    NB = N // 8
    hb1 = hb1_ref[...]

    ids = lax.broadcasted_iota(jnp.int32, (BP, N), 0)
    oht = jnp.where(ids == memrow_ref[...], f32(1.0), f32(0.0))
    gsum = jax.lax.dot(oht, hb1, precision=HIGH, preferred_element_type=f32)

    bmax_ref[...] = jnp.max(hb1.reshape(NB, 8, D), axis=1)
    neg_inf = f32(-jnp.inf)
    lo = memblk_ref[:, 0:1]
    hi = memblk_ref[:, 7:8]

    def p1_body(b, _):
        maskp = jnp.logical_and(lo == b, hi == b)
        seg = jnp.where(maskp, bmax_ref[...], neg_inf)
        gmax_ref[pl.ds(b, 1), :] = jnp.max(seg, axis=0, keepdims=True)
        return 0
    lax.fori_loop(0, B, p1_body, 0)

    def p2_body(i, _):
        blo = mems_ref[8 * i]
        bhi = mems_ref[8 * i + 7]
        @pl.when(blo != bhi)
        def _():
            for k in range(8):
                r = 8 * i + k
                m = mems_ref[r]
                row = hb1_ref[pl.ds(r, 1), :]
                cur = gmax_ref[pl.ds(m, 1), :]
                gmax_ref[pl.ds(m, 1), :] = jnp.maximum(cur, row)
        return 0
    lax.fori_loop(0, NB, p2_body, 0)
    gmax = gmax_ref[...]

    ro = jnp.tanh(jnp.concatenate([gsum, gmax], axis=1))
    le = jax.lax.dot(ro, wd2e_ref[...], precision=HIGH,
                     preferred_element_type=f32) + bd2e_ref[...]
    lo = jax.lax.dot(ro, wd2o_ref[...], precision=HIGH,
                     preferred_element_type=f32) + bd2o_ref[...]
    mx = jnp.maximum(le, lo)
    ee = jnp.exp(le - mx)
    eo = jnp.exp(lo - mx)
    inv = 1.0 / (ee + eo)
    p0_ref[...] = ee * inv
    p1_ref[...] = eo * inv


_readout_call = pl.pallas_call(
    _readout_body,
    in_specs=[
        pl.BlockSpec(memory_space=pltpu.VMEM),
        pl.BlockSpec(memory_space=pltpu.VMEM),
        pl.BlockSpec(memory_space=pltpu.SMEM),
        pl.BlockSpec(memory_space=pltpu.VMEM),
        pl.BlockSpec(memory_space=pltpu.VMEM),
        pl.BlockSpec(memory_space=pltpu.VMEM),
        pl.BlockSpec(memory_space=pltpu.VMEM),
        pl.BlockSpec(memory_space=pltpu.VMEM),
    ],
    out_shape=[
        jax.ShapeDtypeStruct((BP, NT), jnp.float32),
        jax.ShapeDtypeStruct((BP, NT), jnp.float32),
    ],
    scratch_shapes=[
        pltpu.VMEM((BP, D), jnp.float32),
        pltpu.VMEM((N // 8, D), jnp.float32),
    ],
)


def kernel(x, W1s, W1n, b1, g1, be1, W2s, W2n, b2, g2, be2, Wd1, bd1, g3,
           be3, Wd2, bd2, edge_index, membership):
    src4d = edge_index[0].reshape(NW, NBLK, CPB, CHUNK)
    dst4d = edge_index[1].reshape(NW, NBLK, CPB, CHUNK)
    zeros = jnp.zeros((N, D), jnp.float32)
    agg0, agg1 = _edge_agg_call()(zeros, src4d, dst4d, x)

    xw = _xw_call(x, W1s)
    hb1 = _conv_call(xw, agg0, agg1, W1n, b1.reshape(1, D),
                     g1.reshape(1, D), be1.reshape(1, D))
    memrow = membership.reshape(1, N)
    memblk = membership.reshape(N // 8, 8)
    p0, p1 = _readout_call(
        memrow, memblk, membership, Wd2[:, 0::2], Wd2[:, 1::2],
        bd2[0::2].reshape(1, NT), bd2[1::2].reshape(1, NT), hb1)
    return jnp.stack([p0[:B], p1[:B]], axis=-1)
